# speedup vs baseline: 1.1796x; 1.1796x over previous
import jax
import jax.numpy as jnp
from jax.experimental import pallas as pl
from jax.experimental.pallas import tpu as pltpu

_MINIMUM = 0.3
_MAXIMUM = 0.9


def _body(l_ref, m_ref, out_ref):
    b = pl.program_id(0)
    c = pl.program_id(1)

    @pl.when((b == 0) & (c == 0))
    def _init():
        out_ref[0, 0] = 0.0

    l = l_ref[0, 0]
    m = m_ref[0, 0]
    box = jnp.sum(m, axis=(1, 2))
    act = jnp.sum(m * l[None, :, :], axis=(1, 2))
    over = act - _MAXIMUM * box
    under = _MINIMUM * box - act
    err = (jnp.where(over >= 0, over * over, 0.0)
           + jnp.where(under >= 0, under * under, 0.0))
    out_ref[0, 0] += jnp.sum(err)


def kernel(logits, box_masks):
    B, C, W, H = logits.shape
    N = box_masks.shape[2]
    Cf = C - 1

    out = pl.pallas_call(
        _body,
        grid=(B, Cf),
        in_specs=[
            pl.BlockSpec((1, 1, W, H), lambda b, c: (b, c + 1, 0, 0)),
            pl.BlockSpec((1, 1, N, W, H), lambda b, c: (b, c + 1, 0, 0, 0)),
        ],
        out_specs=pl.BlockSpec(memory_space=pltpu.SMEM),
        out_shape=jax.ShapeDtypeStruct((1, 1), jnp.float32),
    )(logits, box_masks)
    return out[0, 0] / float(Cf * W * H)
